# baseline (speedup 1.0000x reference)
import functools

import jax
import jax.numpy as jnp
from jax.experimental import pallas as pl
from jax.experimental.pallas import tpu as pltpu
from jax.experimental.shard_map import shard_map
from jax.sharding import Mesh, PartitionSpec as P


def _sage_layer_body(adj_ref, xa_ref, xs_ref, ws_ref, wn_ref, b_ref, out_ref,
                     *, feat, apply_relu):
    a = adj_ref[...].astype(jnp.bfloat16)
    prod = jnp.dot(a, xa_ref[...], preferred_element_type=jnp.float32)
    s = prod[:, :feat]
    deg = jnp.clip(prod[:, feat:feat + 1], 1e-6, None)
    neigh = s / deg
    h = (jnp.dot(xs_ref[...], ws_ref[...], preferred_element_type=jnp.float32,
                 precision=jax.lax.Precision.HIGHEST)
         + jnp.dot(neigh, wn_ref[...], preferred_element_type=jnp.float32,
                   precision=jax.lax.Precision.HIGHEST)
         + b_ref[...])
    if apply_relu:
        h = jnp.maximum(h, 0.0)
    out_ref[...] = h


def _pick_bm(n):
    for c in (400, 256, 200, 128, 80, 64, 40, 32, 16, 8):
        if n % c == 0:
            return c
    return n


def _sage_layer(adj, x_nbr, x_self, w, b, apply_relu):
    nrows, ncols = adj.shape
    feat = x_nbr.shape[1]
    bm = _pick_bm(nrows)
    xa = jnp.concatenate(
        [x_nbr.astype(jnp.bfloat16), jnp.ones((ncols, 1), jnp.bfloat16)],
        axis=1)
    ws = w[:feat]
    wn = w[feat:]
    b2 = b.reshape(1, feat)
    body = functools.partial(_sage_layer_body, feat=feat, apply_relu=apply_relu)
    return pl.pallas_call(
        body,
        grid=(nrows // bm,),
        in_specs=[
            pl.BlockSpec((bm, ncols), lambda i: (i, 0)),
            pl.BlockSpec((ncols, feat + 1), lambda i: (0, 0)),
            pl.BlockSpec((bm, feat), lambda i: (i, 0)),
            pl.BlockSpec((feat, feat), lambda i: (0, 0)),
            pl.BlockSpec((feat, feat), lambda i: (0, 0)),
            pl.BlockSpec((1, feat), lambda i: (0, 0)),
        ],
        out_specs=pl.BlockSpec((bm, feat), lambda i: (i, 0)),
        out_shape=jax.ShapeDtypeStruct((nrows, feat), jnp.float32),
        compiler_params=pltpu.CompilerParams(
            dimension_semantics=("arbitrary",),
        ),
    )(adj, xa, x_self, ws, wn, b2)


def _two_layers(adj, fts, W1, b1, W2, b2):
    h = _sage_layer(adj, fts, fts, W1, b1, apply_relu=True)
    return _sage_layer(adj, h, h, W2, b2, apply_relu=False)


def kernel(fts, adj, W1, b1, W2, b2):
    devs = jax.devices()
    n = adj.shape[0]
    nd = len(devs)
    if nd < 2 or n % nd != 0:
        return _two_layers(adj, fts, W1, b1, W2, b2)
    mesh = Mesh(devs, ("x",))

    rows_per = n // nd

    def _sharded(fts, adj_rows, W1, b1, W2, b2):
        base = jax.lax.axis_index("x") * rows_per
        fts_self = jax.lax.dynamic_slice_in_dim(fts, base, rows_per, axis=0)
        h_local = _sage_layer(adj_rows, fts, fts_self, W1, b1,
                              apply_relu=True)
        h = jax.lax.all_gather(h_local, "x", axis=0, tiled=True)
        return _sage_layer(adj_rows, h, h_local, W2, b2, apply_relu=False)

    return shard_map(
        _sharded, mesh=mesh,
        in_specs=(P(), P("x", None), P(), P(), P(), P()),
        out_specs=P("x", None),
        check_rep=False,
    )(fts, adj, W1, b1, W2, b2)


# speedup vs baseline: 3.0093x; 3.0093x over previous
import functools

import jax
import jax.numpy as jnp
from jax.experimental import pallas as pl
from jax.experimental.pallas import tpu as pltpu


def _sage_layer_body(adj_ref, xa_ref, xs_ref, ws_ref, wn_ref, b_ref, out_ref,
                     *, feat, apply_relu):
    a = adj_ref[...].astype(jnp.bfloat16)
    prod = jnp.dot(a, xa_ref[...], preferred_element_type=jnp.float32)
    s = prod[:, :feat]
    deg = jnp.clip(prod[:, feat:feat + 1], 1e-6, None)
    neigh = s / deg
    h = (jnp.dot(xs_ref[...], ws_ref[...], preferred_element_type=jnp.float32,
                 precision=jax.lax.Precision.HIGHEST)
         + jnp.dot(neigh, wn_ref[...], preferred_element_type=jnp.float32,
                   precision=jax.lax.Precision.HIGHEST)
         + b_ref[...])
    if apply_relu:
        h = jnp.maximum(h, 0.0)
    out_ref[...] = h


def _pick_bm(n):
    for c in (400, 256, 200, 128, 80, 64, 40, 32, 16, 8):
        if n % c == 0:
            return c
    return n


def _sage_layer(adj, x_nbr, x_self, w, b, apply_relu):
    nrows, ncols = adj.shape
    feat = x_nbr.shape[1]
    bm = _pick_bm(nrows)
    xa = jnp.concatenate(
        [x_nbr.astype(jnp.bfloat16), jnp.ones((ncols, 1), jnp.bfloat16)],
        axis=1)
    ws = w[:feat]
    wn = w[feat:]
    b2 = b.reshape(1, feat)
    body = functools.partial(_sage_layer_body, feat=feat, apply_relu=apply_relu)
    return pl.pallas_call(
        body,
        grid=(nrows // bm,),
        in_specs=[
            pl.BlockSpec((bm, ncols), lambda i: (i, 0)),
            pl.BlockSpec((ncols, feat + 1), lambda i: (0, 0)),
            pl.BlockSpec((bm, feat), lambda i: (i, 0)),
            pl.BlockSpec((feat, feat), lambda i: (0, 0)),
            pl.BlockSpec((feat, feat), lambda i: (0, 0)),
            pl.BlockSpec((1, feat), lambda i: (0, 0)),
        ],
        out_specs=pl.BlockSpec((bm, feat), lambda i: (i, 0)),
        out_shape=jax.ShapeDtypeStruct((nrows, feat), jnp.float32),
        compiler_params=pltpu.CompilerParams(
            dimension_semantics=("arbitrary",),
        ),
    )(adj, xa, x_self, ws, wn, b2)


def kernel(fts, adj, W1, b1, W2, b2):
    h = _sage_layer(adj, fts, fts, W1, b1, apply_relu=True)
    return _sage_layer(adj, h, h, W2, b2, apply_relu=False)
